# baseline (speedup 1.0000x reference)
import functools

import jax
import jax.numpy as jnp
import numpy as np
from jax import lax
from jax.experimental import pallas as pl
from jax.experimental.pallas import tpu as pltpu
from jax.experimental.pallas import tpu_sc as plsc

_H = _W = 11
_HW = _H * _W
_C = 128
_E = _HW * _C
_B = 4096


def _spiral_perm() -> np.ndarray:
    cen = _H // 2
    pos = [(cen, cen)]
    for r in range(1, cen + 1):
        pos += [(cen - r, w) for w in range(cen - r + 1, cen + r + 1)]
        pos += [(h, cen + r) for h in range(cen - r + 1, cen + r + 1)]
        pos += [(cen + r, w) for w in range(cen - r, cen + r)]
        pos += [(h, cen - r) for h in range(cen - r, cen + r)]
    return np.array([h * _W + w for h, w in pos], dtype=np.int32)


_PERM_PAD = np.zeros((128,), dtype=np.int32)
_PERM_PAD[:_HW] = _spiral_perm()

_NC, _NS = 2, 16
_NW = _NC * _NS
_NB = _B // _NW


def _body(x_hbm, perm_hbm, out_hbm, ptab, inbuf, outbuf):
    pltpu.sync_copy(perm_hbm, ptab)
    wid = lax.axis_index("s") * _NC + lax.axis_index("c")
    iot = lax.iota(jnp.int32, 16)
    pvs, ovs, ms = [], [], []
    for kt in range(8):
        k16 = kt * 16 + iot
        pvs.append(ptab[pl.ds(kt * 16, 16)])
        ovs.append(k16 * _C)
        ms.append(k16 < _HW)

    def bloop(i, carry):
        bidx = wid * _NB + i
        pltpu.sync_copy(x_hbm.at[bidx], inbuf)

        def cloop(c, carry2):
            base = c * _HW
            for kt in range(8):
                vals = plsc.load_gather(inbuf, [pvs[kt] + base])
                plsc.store_scatter(outbuf, [ovs[kt] + c], vals, mask=ms[kt])
            return carry2

        lax.fori_loop(0, _C, cloop, 0)
        pltpu.sync_copy(outbuf, out_hbm.at[bidx])
        return carry

    lax.fori_loop(0, _NB, bloop, 0)


@jax.jit
def kernel(x):
    xr = x.reshape(_B, _E)
    perm = jnp.asarray(_PERM_PAD)
    mesh = plsc.VectorSubcoreMesh(core_axis_name="c", subcore_axis_name="s",
                                  num_cores=_NC)
    out = pl.kernel(
        _body,
        mesh=mesh,
        compiler_params=pltpu.CompilerParams(needs_layout_passes=False),
        out_type=jax.ShapeDtypeStruct((_B, _E), jnp.float32),
        scratch_types=[
            pltpu.VMEM((128,), jnp.int32),
            pltpu.VMEM((_E,), jnp.float32),
            pltpu.VMEM((_E,), jnp.float32),
        ],
    )(xr, perm)
    return out.reshape(_B, _HW, _C)


# speedup vs baseline: 1.3742x; 1.3742x over previous
import jax
import jax.numpy as jnp
import numpy as np
from jax import lax
from jax.experimental import pallas as pl
from jax.experimental.pallas import tpu as pltpu
from jax.experimental.pallas import tpu_sc as plsc

_H = _W = 11
_HW = _H * _W
_C = 128
_E = _HW * _C
_B = 4096


def _spiral_perm() -> np.ndarray:
    cen = _H // 2
    pos = [(cen, cen)]
    for r in range(1, cen + 1):
        pos += [(cen - r, w) for w in range(cen - r + 1, cen + r + 1)]
        pos += [(h, cen + r) for h in range(cen - r + 1, cen + r + 1)]
        pos += [(cen + r, w) for w in range(cen - r, cen + r)]
        pos += [(h, cen - r) for h in range(cen - r, cen + r)]
    return np.array([h * _W + w for h, w in pos], dtype=np.int32)


_PERM_PAD = np.zeros((128,), dtype=np.int32)
_PERM_PAD[:_HW] = _spiral_perm()

_NC, _NS = 2, 16
_NW = _NC * _NS
_NB = _B // _NW


def _body(x_hbm, perm_hbm, out_hbm, ptab, in0, in1, out0, out1,
          si0, si1, so0, so1):
    pltpu.sync_copy(perm_hbm, ptab)
    wid = lax.axis_index("s") * _NC + lax.axis_index("c")
    base = wid * _NB
    ins, outs, sis, sos = (in0, in1), (out0, out1), (si0, si1), (so0, so1)

    iot = lax.iota(jnp.int32, 16)
    pvs, ovs, ms = [], [], []
    for kt in range(8):
        k16 = kt * 16 + iot
        pvs.append(ptab[pl.ds(kt * 16, 16)])
        ovs.append(k16 * _C)
        ms.append((k16 < _HW) if kt == 7 else None)

    pltpu.async_copy(x_hbm.at[base], in0, si0)
    pltpu.async_copy(x_hbm.at[base + 1], in1, si1)

    def shuffle(inref, outref):
        @plsc.parallel_loop(0, _C, step=1, unroll=4)
        def cbody(c):
            basec = c * _HW
            for kt in range(8):
                vals = plsc.load_gather(inref, [pvs[kt] + basec])
                plsc.store_scatter(outref, [ovs[kt] + c], vals, mask=ms[kt])

    def gloop(g, carry):
        for s in range(2):
            i = 2 * g + s
            pltpu.make_async_copy(x_hbm.at[base + i], ins[s], sis[s]).wait()

            @pl.when(g > 0)
            def _wait_out():
                pltpu.make_async_copy(
                    outs[s], out_hbm.at[base + i - 2], sos[s]).wait()

            shuffle(ins[s], outs[s])
            pltpu.async_copy(outs[s], out_hbm.at[base + i], sos[s])

            @pl.when(i + 2 < _NB)
            def _next_in():
                pltpu.async_copy(x_hbm.at[base + i + 2], ins[s], sis[s])
        return carry

    lax.fori_loop(0, _NB // 2, gloop, 0)
    pltpu.make_async_copy(out0, out_hbm.at[base + _NB - 2], so0).wait()
    pltpu.make_async_copy(out1, out_hbm.at[base + _NB - 1], so1).wait()


@jax.jit
def kernel(x):
    xr = x.reshape(_B, _E)
    perm = jnp.asarray(_PERM_PAD)
    mesh = plsc.VectorSubcoreMesh(core_axis_name="c", subcore_axis_name="s",
                                  num_cores=_NC)
    out = pl.kernel(
        _body,
        mesh=mesh,
        compiler_params=pltpu.CompilerParams(needs_layout_passes=False),
        out_type=jax.ShapeDtypeStruct((_B, _E), jnp.float32),
        scratch_types=[
            pltpu.VMEM((128,), jnp.int32),
            pltpu.VMEM((_E,), jnp.float32),
            pltpu.VMEM((_E,), jnp.float32),
            pltpu.VMEM((_E,), jnp.float32),
            pltpu.VMEM((_E,), jnp.float32),
            pltpu.SemaphoreType.DMA,
            pltpu.SemaphoreType.DMA,
            pltpu.SemaphoreType.DMA,
            pltpu.SemaphoreType.DMA,
        ],
    )(xr, perm)
    return out.reshape(_B, _HW, _C)


# speedup vs baseline: 1.3880x; 1.0100x over previous
import jax
import jax.numpy as jnp
import numpy as np
from jax import lax
from jax.experimental import pallas as pl
from jax.experimental.pallas import tpu as pltpu
from jax.experimental.pallas import tpu_sc as plsc

_H = _W = 11
_HW = _H * _W
_C = 128
_E = _HW * _C
_B = 4096


def _spiral_perm() -> np.ndarray:
    cen = _H // 2
    pos = [(cen, cen)]
    for r in range(1, cen + 1):
        pos += [(cen - r, w) for w in range(cen - r + 1, cen + r + 1)]
        pos += [(h, cen + r) for h in range(cen - r + 1, cen + r + 1)]
        pos += [(cen + r, w) for w in range(cen - r, cen + r)]
        pos += [(h, cen - r) for h in range(cen - r, cen + r)]
    return np.array([h * _W + w for h, w in pos], dtype=np.int32)


_PERM_PAD = np.zeros((128,), dtype=np.int32)
_PERM_PAD[:_HW] = _spiral_perm()

_NC, _NS = 2, 16
_NW = _NC * _NS
_NB = _B // _NW


_PERM_LIST = [int(v) for v in _spiral_perm()]


def _body(x_hbm, out_hbm, in0, in1, out0, out1,
          si0, si1, so0, so1):
    wid = lax.axis_index("s") * _NC + lax.axis_index("c")
    base = wid * _NB
    ins, outs, sis, sos = (in0, in1), (out0, out1), (si0, si1), (so0, so1)

    iotav = lax.iota(jnp.int32, 16) * _HW

    pltpu.async_copy(x_hbm.at[base], in0, si0)
    pltpu.async_copy(x_hbm.at[base + 1], in1, si1)

    def shuffle(inref, outref):
        for k in range(_HW):
            pk = _PERM_LIST[k]
            for ct in range(8):
                vals = plsc.load_gather(inref, [iotav + (ct * 16 * _HW + pk)])
                outref[pl.ds(k * _C + ct * 16, 16)] = vals

    def gloop(g, carry):
        for s in range(2):
            i = 2 * g + s
            pltpu.make_async_copy(x_hbm.at[base + i], ins[s], sis[s]).wait()

            @pl.when(g > 0)
            def _wait_out():
                pltpu.make_async_copy(
                    outs[s], out_hbm.at[base + i - 2], sos[s]).wait()

            shuffle(ins[s], outs[s])
            pltpu.async_copy(outs[s], out_hbm.at[base + i], sos[s])

            @pl.when(i + 2 < _NB)
            def _next_in():
                pltpu.async_copy(x_hbm.at[base + i + 2], ins[s], sis[s])
        return carry

    lax.fori_loop(0, _NB // 2, gloop, 0)
    pltpu.make_async_copy(out0, out_hbm.at[base + _NB - 2], so0).wait()
    pltpu.make_async_copy(out1, out_hbm.at[base + _NB - 1], so1).wait()


@jax.jit
def kernel(x):
    xr = x.reshape(_B, _E)
    mesh = plsc.VectorSubcoreMesh(core_axis_name="c", subcore_axis_name="s",
                                  num_cores=_NC)
    out = pl.kernel(
        _body,
        mesh=mesh,
        compiler_params=pltpu.CompilerParams(needs_layout_passes=False),
        out_type=jax.ShapeDtypeStruct((_B, _E), jnp.float32),
        scratch_types=[
            pltpu.VMEM((_E,), jnp.float32),
            pltpu.VMEM((_E,), jnp.float32),
            pltpu.VMEM((_E,), jnp.float32),
            pltpu.VMEM((_E,), jnp.float32),
            pltpu.SemaphoreType.DMA,
            pltpu.SemaphoreType.DMA,
            pltpu.SemaphoreType.DMA,
            pltpu.SemaphoreType.DMA,
        ],
    )(xr)
    return out.reshape(_B, _HW, _C)
